# Initial kernel scaffold; baseline (speedup 1.0000x reference)
import jax
import jax.numpy as jnp
from jax.experimental import pallas as pl


def kernel(img, lut):
    raise NotImplementedError("write your pallas kernel here")


# speedup vs baseline: 595.2203x; 595.2203x over previous
import jax
import jax.numpy as jnp
from jax import lax
from jax.experimental import pallas as pl
from jax.experimental.pallas import tpu as pltpu
from jax.experimental.pallas import tpu_sc as plsc

_B, _C = 4, 3
_GRID = 33
_NPIX = 512 * 512
_NLUT = _GRID * _GRID * _GRID
_NLUT_PAD = 35944
_NC, _NS = 2, 16
_NW = _NC * _NS
_TPB = _NW // _B
_PPT = _NPIX // _TPB
_CHUNK = 2048
_L = 16


def _sc_body(img_hbm, lut_hbm, out_hbm,
             lut0, lut1, lut2, xb, yb, zb, o0, o1, o2):
    wid = lax.axis_index("s") * _NC + lax.axis_index("c")
    b = wid // _TPB
    sub = wid % _TPB
    lbase = b * (_C * _NLUT_PAD)
    pltpu.sync_copy(lut_hbm.at[pl.ds(pl.multiple_of(lbase, 8), _NLUT_PAD)], lut0)
    pltpu.sync_copy(lut_hbm.at[pl.ds(pl.multiple_of(lbase + _NLUT_PAD, 8), _NLUT_PAD)], lut1)
    pltpu.sync_copy(lut_hbm.at[pl.ds(pl.multiple_of(lbase + 2 * _NLUT_PAD, 8), _NLUT_PAD)], lut2)
    base = b * (_C * _NPIX) + sub * _PPT

    @pl.loop(0, _PPT // _CHUNK)
    def _chunk(t):
        off = pl.multiple_of(base + t * _CHUNK, _CHUNK)
        pltpu.sync_copy(img_hbm.at[pl.ds(off, _CHUNK)], xb)
        pltpu.sync_copy(img_hbm.at[pl.ds(off + _NPIX, _CHUNK)], yb)
        pltpu.sync_copy(img_hbm.at[pl.ds(off + 2 * _NPIX, _CHUNK)], zb)

        @pl.loop(0, _CHUNK // _L)
        def _grp(g):
            s = pl.multiple_of(g * _L, _L)

            def coord(v):
                gg = v * 2.0 - 1.0
                cc = jnp.clip((gg + 1.0) * 0.5 * (_GRID - 1.0),
                              0.0, _GRID - 1.0)
                i0 = cc.astype(jnp.int32)
                w = cc - i0.astype(jnp.float32)
                d = jnp.minimum(i0 + 1, _GRID - 1) - i0
                return i0, d, w

            x0, dx, wx = coord(xb[pl.ds(s, _L)])
            y0, dy, wy = coord(yb[pl.ds(s, _L)])
            z0, dz, wz = coord(zb[pl.ds(s, _L)])
            i000 = (z0 * _GRID + y0) * _GRID + x0
            dy = dy * _GRID
            dz = dz * (_GRID * _GRID)
            i010 = i000 + dy
            i100 = i000 + dz
            i110 = i100 + dy

            for ref, ob in ((lut0, o0), (lut1, o1), (lut2, o2)):
                c000 = plsc.load_gather(ref, [i000])
                c001 = plsc.load_gather(ref, [i000 + dx])
                c010 = plsc.load_gather(ref, [i010])
                c011 = plsc.load_gather(ref, [i010 + dx])
                c100 = plsc.load_gather(ref, [i100])
                c101 = plsc.load_gather(ref, [i100 + dx])
                c110 = plsc.load_gather(ref, [i110])
                c111 = plsc.load_gather(ref, [i110 + dx])
                c00 = c000 + wx * (c001 - c000)
                c01 = c010 + wx * (c011 - c010)
                c10 = c100 + wx * (c101 - c100)
                c11 = c110 + wx * (c111 - c110)
                c0 = c00 + wy * (c01 - c00)
                c1 = c10 + wy * (c11 - c10)
                ob[pl.ds(s, _L)] = c0 + wz * (c1 - c0)

        pltpu.sync_copy(o0, out_hbm.at[pl.ds(off, _CHUNK)])
        pltpu.sync_copy(o1, out_hbm.at[pl.ds(off + _NPIX, _CHUNK)])
        pltpu.sync_copy(o2, out_hbm.at[pl.ds(off + 2 * _NPIX, _CHUNK)])


def kernel(img, lut):
    imgf = img.reshape(_B * _C * _NPIX)
    lutf = lut.reshape(_B, _C, _NLUT)
    lutp = jnp.pad(lutf, ((0, 0), (0, 0), (0, _NLUT_PAD - _NLUT)))
    lutp = lutp.reshape(_B * _C * _NLUT_PAD)
    vm = lambda n: pltpu.VMEM((n,), jnp.float32)
    k = pl.kernel(
        _sc_body,
        out_type=jax.ShapeDtypeStruct((_B * _C * _NPIX,), jnp.float32),
        mesh=plsc.VectorSubcoreMesh(core_axis_name="c", subcore_axis_name="s"),
        scratch_types=[vm(_NLUT_PAD), vm(_NLUT_PAD), vm(_NLUT_PAD),
                       vm(_CHUNK), vm(_CHUNK), vm(_CHUNK),
                       vm(_CHUNK), vm(_CHUNK), vm(_CHUNK)],
        compiler_params=pltpu.CompilerParams(needs_layout_passes=False),
    )
    return k(imgf, lutp).reshape(_B, _C, 512, 512)


# speedup vs baseline: 707.0648x; 1.1879x over previous
import jax
import jax.numpy as jnp
from jax import lax
from jax.experimental import pallas as pl
from jax.experimental.pallas import tpu as pltpu
from jax.experimental.pallas import tpu_sc as plsc

_B, _C = 4, 3
_GRID = 33
_NPIX = 512 * 512
_NLUT = _GRID * _GRID * _GRID
_NLUT_PAD = 35944
_NC, _NS = 2, 16
_NW = _NC * _NS
_TPB = _NW // _B
_PPT = _NPIX // _TPB
_CHUNK = 2048
_L = 16


def _sc_body(img_hbm, lut_hbm, out_hbm,
             lut0, lut1, lut2, xb, yb, zb, o0, o1, o2):
    wid = lax.axis_index("s") * _NC + lax.axis_index("c")
    b = wid // _TPB
    sub = wid % _TPB
    lbase = b * (_C * _NLUT_PAD)
    pltpu.sync_copy(lut_hbm.at[pl.ds(pl.multiple_of(lbase, 8), _NLUT_PAD)], lut0)
    pltpu.sync_copy(lut_hbm.at[pl.ds(pl.multiple_of(lbase + _NLUT_PAD, 8), _NLUT_PAD)], lut1)
    pltpu.sync_copy(lut_hbm.at[pl.ds(pl.multiple_of(lbase + 2 * _NLUT_PAD, 8), _NLUT_PAD)], lut2)
    base = b * (_C * _NPIX) + sub * _PPT

    @pl.loop(0, _PPT // _CHUNK)
    def _chunk(t):
        off = pl.multiple_of(base + t * _CHUNK, _CHUNK)
        pltpu.sync_copy(img_hbm.at[pl.ds(off, _CHUNK)], xb)
        pltpu.sync_copy(img_hbm.at[pl.ds(off + _NPIX, _CHUNK)], yb)
        pltpu.sync_copy(img_hbm.at[pl.ds(off + 2 * _NPIX, _CHUNK)], zb)

        @plsc.parallel_loop(0, _CHUNK, step=_L, unroll=4)
        def _grp(g):
            s = pl.multiple_of(g, _L)

            def coord(v):
                gg = v * 2.0 - 1.0
                cc = jnp.clip((gg + 1.0) * 0.5 * (_GRID - 1.0),
                              0.0, _GRID - 1.0)
                i0 = cc.astype(jnp.int32)
                w = cc - i0.astype(jnp.float32)
                d = jnp.minimum(i0 + 1, _GRID - 1) - i0
                return i0, d, w

            x0, dx, wx = coord(xb[pl.ds(s, _L)])
            y0, dy, wy = coord(yb[pl.ds(s, _L)])
            z0, dz, wz = coord(zb[pl.ds(s, _L)])
            i000 = (z0 * _GRID + y0) * _GRID + x0
            dy = dy * _GRID
            dz = dz * (_GRID * _GRID)
            i010 = i000 + dy
            i100 = i000 + dz
            i110 = i100 + dy

            for ref, ob in ((lut0, o0), (lut1, o1), (lut2, o2)):
                c000 = plsc.load_gather(ref, [i000])
                c001 = plsc.load_gather(ref, [i000 + dx])
                c010 = plsc.load_gather(ref, [i010])
                c011 = plsc.load_gather(ref, [i010 + dx])
                c100 = plsc.load_gather(ref, [i100])
                c101 = plsc.load_gather(ref, [i100 + dx])
                c110 = plsc.load_gather(ref, [i110])
                c111 = plsc.load_gather(ref, [i110 + dx])
                c00 = c000 + wx * (c001 - c000)
                c01 = c010 + wx * (c011 - c010)
                c10 = c100 + wx * (c101 - c100)
                c11 = c110 + wx * (c111 - c110)
                c0 = c00 + wy * (c01 - c00)
                c1 = c10 + wy * (c11 - c10)
                ob[pl.ds(s, _L)] = c0 + wz * (c1 - c0)

        pltpu.sync_copy(o0, out_hbm.at[pl.ds(off, _CHUNK)])
        pltpu.sync_copy(o1, out_hbm.at[pl.ds(off + _NPIX, _CHUNK)])
        pltpu.sync_copy(o2, out_hbm.at[pl.ds(off + 2 * _NPIX, _CHUNK)])


def kernel(img, lut):
    imgf = img.reshape(_B * _C * _NPIX)
    lutf = lut.reshape(_B, _C, _NLUT)
    lutp = jnp.pad(lutf, ((0, 0), (0, 0), (0, _NLUT_PAD - _NLUT)))
    lutp = lutp.reshape(_B * _C * _NLUT_PAD)
    vm = lambda n: pltpu.VMEM((n,), jnp.float32)
    k = pl.kernel(
        _sc_body,
        out_type=jax.ShapeDtypeStruct((_B * _C * _NPIX,), jnp.float32),
        mesh=plsc.VectorSubcoreMesh(core_axis_name="c", subcore_axis_name="s"),
        scratch_types=[vm(_NLUT_PAD), vm(_NLUT_PAD), vm(_NLUT_PAD),
                       vm(_CHUNK), vm(_CHUNK), vm(_CHUNK),
                       vm(_CHUNK), vm(_CHUNK), vm(_CHUNK)],
        compiler_params=pltpu.CompilerParams(needs_layout_passes=False),
    )
    return k(imgf, lutp).reshape(_B, _C, 512, 512)


# speedup vs baseline: 813.9721x; 1.1512x over previous
import jax
import jax.numpy as jnp
from jax import lax
from jax.experimental import pallas as pl
from jax.experimental.pallas import tpu as pltpu
from jax.experimental.pallas import tpu_sc as plsc

_B, _C = 4, 3
_GRID = 33
_NPIX = 512 * 512
_NLUT = _GRID * _GRID * _GRID
_NLUT_PAD = 35944
_NC, _NS = 2, 16
_NW = _NC * _NS
_TPB = _NW // _B
_PPT = _NPIX // _TPB
_CHUNK = 2048
_L = 16


def _sc_body(img_hbm, lut_hbm, out_hbm,
             lut0, lut1, lut2, xb, yb, zb, o0, o1, o2):
    wid = lax.axis_index("s") * _NC + lax.axis_index("c")
    b = wid // _TPB
    sub = wid % _TPB
    lbase = b * (_C * _NLUT_PAD)
    pltpu.sync_copy(lut_hbm.at[pl.ds(pl.multiple_of(lbase, 8), _NLUT_PAD)], lut0)
    pltpu.sync_copy(lut_hbm.at[pl.ds(pl.multiple_of(lbase + _NLUT_PAD, 8), _NLUT_PAD)], lut1)
    pltpu.sync_copy(lut_hbm.at[pl.ds(pl.multiple_of(lbase + 2 * _NLUT_PAD, 8), _NLUT_PAD)], lut2)
    base = b * (_C * _NPIX) + sub * _PPT

    @pl.loop(0, _PPT // _CHUNK)
    def _chunk(t):
        off = pl.multiple_of(base + t * _CHUNK, _CHUNK)
        pltpu.sync_copy(img_hbm.at[pl.ds(off, _CHUNK)], xb)
        pltpu.sync_copy(img_hbm.at[pl.ds(off + _NPIX, _CHUNK)], yb)
        pltpu.sync_copy(img_hbm.at[pl.ds(off + 2 * _NPIX, _CHUNK)], zb)

        @plsc.parallel_loop(0, _CHUNK, step=_L, unroll=8)
        def _grp(g):
            s = pl.multiple_of(g, _L)

            def coord(v):
                gg = v * 2.0 - 1.0
                cc = jnp.clip((gg + 1.0) * 0.5 * (_GRID - 1.0),
                              0.0, _GRID - 1.0)
                i0 = cc.astype(jnp.int32)
                w = cc - i0.astype(jnp.float32)
                d = jnp.minimum(i0 + 1, _GRID - 1) - i0
                return i0, d, w

            x0, dx, wx = coord(xb[pl.ds(s, _L)])
            y0, dy, wy = coord(yb[pl.ds(s, _L)])
            z0, dz, wz = coord(zb[pl.ds(s, _L)])
            i000 = (z0 * _GRID + y0) * _GRID + x0
            dy = dy * _GRID
            dz = dz * (_GRID * _GRID)
            i010 = i000 + dy
            i100 = i000 + dz
            i110 = i100 + dy

            for ref, ob in ((lut0, o0), (lut1, o1), (lut2, o2)):
                c000 = plsc.load_gather(ref, [i000])
                c001 = plsc.load_gather(ref, [i000 + dx])
                c010 = plsc.load_gather(ref, [i010])
                c011 = plsc.load_gather(ref, [i010 + dx])
                c100 = plsc.load_gather(ref, [i100])
                c101 = plsc.load_gather(ref, [i100 + dx])
                c110 = plsc.load_gather(ref, [i110])
                c111 = plsc.load_gather(ref, [i110 + dx])
                c00 = c000 + wx * (c001 - c000)
                c01 = c010 + wx * (c011 - c010)
                c10 = c100 + wx * (c101 - c100)
                c11 = c110 + wx * (c111 - c110)
                c0 = c00 + wy * (c01 - c00)
                c1 = c10 + wy * (c11 - c10)
                ob[pl.ds(s, _L)] = c0 + wz * (c1 - c0)

        pltpu.sync_copy(o0, out_hbm.at[pl.ds(off, _CHUNK)])
        pltpu.sync_copy(o1, out_hbm.at[pl.ds(off + _NPIX, _CHUNK)])
        pltpu.sync_copy(o2, out_hbm.at[pl.ds(off + 2 * _NPIX, _CHUNK)])


def kernel(img, lut):
    imgf = img.reshape(_B * _C * _NPIX)
    lutf = lut.reshape(_B, _C, _NLUT)
    lutp = jnp.pad(lutf, ((0, 0), (0, 0), (0, _NLUT_PAD - _NLUT)))
    lutp = lutp.reshape(_B * _C * _NLUT_PAD)
    vm = lambda n: pltpu.VMEM((n,), jnp.float32)
    k = pl.kernel(
        _sc_body,
        out_type=jax.ShapeDtypeStruct((_B * _C * _NPIX,), jnp.float32),
        mesh=plsc.VectorSubcoreMesh(core_axis_name="c", subcore_axis_name="s"),
        scratch_types=[vm(_NLUT_PAD), vm(_NLUT_PAD), vm(_NLUT_PAD),
                       vm(_CHUNK), vm(_CHUNK), vm(_CHUNK),
                       vm(_CHUNK), vm(_CHUNK), vm(_CHUNK)],
        compiler_params=pltpu.CompilerParams(needs_layout_passes=False),
    )
    return k(imgf, lutp).reshape(_B, _C, 512, 512)


# speedup vs baseline: 1015.0518x; 1.2470x over previous
import jax
import jax.numpy as jnp
from jax import lax
from jax.experimental import pallas as pl
from jax.experimental.pallas import tpu as pltpu
from jax.experimental.pallas import tpu_sc as plsc

_B, _C = 4, 3
_GRID = 33
_NPIX = 512 * 512
_NLUT = _GRID * _GRID * _GRID
_NLUT_PAD = 35944
_NC, _NS = 2, 16
_NW = _NC * _NS
_TPB = _NW // _B
_PPT = _NPIX // _TPB
_CHUNK = 1024
_NCHUNK = _PPT // _CHUNK
_L = 16


def _sc_body(img_hbm, lut_hbm, out_hbm,
             lut0, lut1, lut2,
             xa, ya, za, oa0, oa1, oa2,
             xb, yb, zb, ob0, ob1, ob2,
             sem_ia, sem_ib, sem_oa, sem_ob):
    wid = lax.axis_index("s") * _NC + lax.axis_index("c")
    b = wid // _TPB
    sub = wid % _TPB
    lbase = b * (_C * _NLUT_PAD)
    pltpu.sync_copy(lut_hbm.at[pl.ds(pl.multiple_of(lbase, 8), _NLUT_PAD)], lut0)
    pltpu.sync_copy(lut_hbm.at[pl.ds(pl.multiple_of(lbase + _NLUT_PAD, 8), _NLUT_PAD)], lut1)
    pltpu.sync_copy(lut_hbm.at[pl.ds(pl.multiple_of(lbase + 2 * _NLUT_PAD, 8), _NLUT_PAD)], lut2)
    base = b * (_C * _NPIX) + sub * _PPT

    set_a = (xa, ya, za, oa0, oa1, oa2, sem_ia, sem_oa)
    set_b = (xb, yb, zb, ob0, ob1, ob2, sem_ib, sem_ob)

    def start_in(bufs, t):
        x, y, z, _, _, _, sem_i, _ = bufs
        off = pl.multiple_of(base + t * _CHUNK, _CHUNK)
        pltpu.async_copy(img_hbm.at[pl.ds(off, _CHUNK)], x, sem_i)
        pltpu.async_copy(img_hbm.at[pl.ds(off + _NPIX, _CHUNK)], y, sem_i)
        pltpu.async_copy(img_hbm.at[pl.ds(off + 2 * _NPIX, _CHUNK)], z, sem_i)

    def wait_in(bufs):
        x, y, z, _, _, _, sem_i, _ = bufs
        for d in (x, y, z):
            pltpu.make_async_copy(img_hbm.at[pl.ds(0, _CHUNK)], d, sem_i).wait()

    def start_out(bufs, t):
        _, _, _, p0, p1, p2, _, sem_o = bufs
        off = pl.multiple_of(base + t * _CHUNK, _CHUNK)
        pltpu.async_copy(p0, out_hbm.at[pl.ds(off, _CHUNK)], sem_o)
        pltpu.async_copy(p1, out_hbm.at[pl.ds(off + _NPIX, _CHUNK)], sem_o)
        pltpu.async_copy(p2, out_hbm.at[pl.ds(off + 2 * _NPIX, _CHUNK)], sem_o)

    def wait_out(bufs):
        _, _, _, p0, p1, p2, _, sem_o = bufs
        for s in (p0, p1, p2):
            pltpu.make_async_copy(s, out_hbm.at[pl.ds(0, _CHUNK)], sem_o).wait()

    def compute(bufs):
        x_r, y_r, z_r, p0, p1, p2, _, _ = bufs

        @plsc.parallel_loop(0, _CHUNK, step=_L, unroll=8)
        def _grp(g):
            s = pl.multiple_of(g, _L)

            def coord(v):
                gg = v * 2.0 - 1.0
                cc = jnp.clip((gg + 1.0) * 0.5 * (_GRID - 1.0),
                              0.0, _GRID - 1.0)
                i0 = cc.astype(jnp.int32)
                w = cc - i0.astype(jnp.float32)
                d = jnp.minimum(i0 + 1, _GRID - 1) - i0
                return i0, d, w

            x0, dx, wx = coord(x_r[pl.ds(s, _L)])
            y0, dy, wy = coord(y_r[pl.ds(s, _L)])
            z0, dz, wz = coord(z_r[pl.ds(s, _L)])
            i000 = (z0 * _GRID + y0) * _GRID + x0
            dy = dy * _GRID
            dz = dz * (_GRID * _GRID)
            i010 = i000 + dy
            i100 = i000 + dz
            i110 = i100 + dy

            for ref, ob in ((lut0, p0), (lut1, p1), (lut2, p2)):
                c000 = plsc.load_gather(ref, [i000])
                c001 = plsc.load_gather(ref, [i000 + dx])
                c010 = plsc.load_gather(ref, [i010])
                c011 = plsc.load_gather(ref, [i010 + dx])
                c100 = plsc.load_gather(ref, [i100])
                c101 = plsc.load_gather(ref, [i100 + dx])
                c110 = plsc.load_gather(ref, [i110])
                c111 = plsc.load_gather(ref, [i110 + dx])
                c00 = c000 + wx * (c001 - c000)
                c01 = c010 + wx * (c011 - c010)
                c10 = c100 + wx * (c101 - c100)
                c11 = c110 + wx * (c111 - c110)
                c0 = c00 + wy * (c01 - c00)
                c1 = c10 + wy * (c11 - c10)
                ob[pl.ds(s, _L)] = c0 + wz * (c1 - c0)

    start_in(set_a, 0)

    @pl.loop(0, _NCHUNK, step=2)
    def _pair(t):
        start_in(set_b, t + 1)
        wait_in(set_a)

        @pl.when(t >= 2)
        def _():
            wait_out(set_a)

        compute(set_a)
        start_out(set_a, t)

        @pl.when(t + 2 < _NCHUNK)
        def _():
            start_in(set_a, t + 2)

        wait_in(set_b)

        @pl.when(t >= 2)
        def _():
            wait_out(set_b)

        compute(set_b)
        start_out(set_b, t + 1)

    wait_out(set_a)
    wait_out(set_b)


def kernel(img, lut):
    imgf = img.reshape(_B * _C * _NPIX)
    lutf = lut.reshape(_B, _C, _NLUT)
    lutp = jnp.pad(lutf, ((0, 0), (0, 0), (0, _NLUT_PAD - _NLUT)))
    lutp = lutp.reshape(_B * _C * _NLUT_PAD)
    vm = lambda n: pltpu.VMEM((n,), jnp.float32)
    k = pl.kernel(
        _sc_body,
        out_type=jax.ShapeDtypeStruct((_B * _C * _NPIX,), jnp.float32),
        mesh=plsc.VectorSubcoreMesh(core_axis_name="c", subcore_axis_name="s"),
        scratch_types=[vm(_NLUT_PAD), vm(_NLUT_PAD), vm(_NLUT_PAD)]
                      + [vm(_CHUNK)] * 12
                      + [pltpu.SemaphoreType.DMA] * 4,
        compiler_params=pltpu.CompilerParams(needs_layout_passes=False),
    )
    return k(imgf, lutp).reshape(_B, _C, 512, 512)


# speedup vs baseline: 1175.7435x; 1.1583x over previous
import jax
import jax.numpy as jnp
from jax import lax
from jax.experimental import pallas as pl
from jax.experimental.pallas import tpu as pltpu
from jax.experimental.pallas import tpu_sc as plsc

_B, _C = 4, 3
_GRID = 33
_NPIX = 512 * 512
_NLUT = _GRID * _GRID * _GRID
_NLUT_PAD = 35944
_NC, _NS = 2, 16
_NW = _NC * _NS
_TPB = _NW // _B
_PPT = _NPIX // _TPB
_CHUNK = 1024
_NCHUNK = _PPT // _CHUNK
_L = 16


_CMAX = 31.999998092651367
_SHIFTS = (0, 1, _GRID, _GRID + 1,
           _GRID * _GRID, _GRID * _GRID + 1,
           _GRID * _GRID + _GRID, _GRID * _GRID + _GRID + 1)
_SLICE = 34816


def _sc_body(img_hbm, lut_hbm, out_hbm,
             lut0, lut1, lut2,
             xa, ya, za, oa0, oa1, oa2,
             xb, yb, zb, ob0, ob1, ob2,
             sem_ia, sem_ib, sem_oa, sem_ob, sem_lut):
    wid = lax.axis_index("s") * _NC + lax.axis_index("c")
    b = wid // _TPB
    sub = wid % _TPB
    lbase = b * (_C * _NLUT_PAD)
    pltpu.async_copy(lut_hbm.at[pl.ds(pl.multiple_of(lbase, 8), _NLUT_PAD)], lut0, sem_lut)
    pltpu.async_copy(lut_hbm.at[pl.ds(pl.multiple_of(lbase + _NLUT_PAD, 8), _NLUT_PAD)], lut1, sem_lut)
    pltpu.async_copy(lut_hbm.at[pl.ds(pl.multiple_of(lbase + 2 * _NLUT_PAD, 8), _NLUT_PAD)], lut2, sem_lut)
    base = b * (_C * _NPIX) + sub * _PPT

    set_a = (xa, ya, za, oa0, oa1, oa2, sem_ia, sem_oa)
    set_b = (xb, yb, zb, ob0, ob1, ob2, sem_ib, sem_ob)

    def start_in(bufs, t):
        x, y, z, _, _, _, sem_i, _ = bufs
        off = pl.multiple_of(base + t * _CHUNK, _CHUNK)
        pltpu.async_copy(img_hbm.at[pl.ds(off, _CHUNK)], x, sem_i)
        pltpu.async_copy(img_hbm.at[pl.ds(off + _NPIX, _CHUNK)], y, sem_i)
        pltpu.async_copy(img_hbm.at[pl.ds(off + 2 * _NPIX, _CHUNK)], z, sem_i)

    def wait_in(bufs):
        x, y, z, _, _, _, sem_i, _ = bufs
        for d in (x, y, z):
            pltpu.make_async_copy(img_hbm.at[pl.ds(0, _CHUNK)], d, sem_i).wait()

    def start_out(bufs, t):
        _, _, _, p0, p1, p2, _, sem_o = bufs
        off = pl.multiple_of(base + t * _CHUNK, _CHUNK)
        pltpu.async_copy(p0, out_hbm.at[pl.ds(off, _CHUNK)], sem_o)
        pltpu.async_copy(p1, out_hbm.at[pl.ds(off + _NPIX, _CHUNK)], sem_o)
        pltpu.async_copy(p2, out_hbm.at[pl.ds(off + 2 * _NPIX, _CHUNK)], sem_o)

    def wait_out(bufs):
        _, _, _, p0, p1, p2, _, sem_o = bufs
        for s in (p0, p1, p2):
            pltpu.make_async_copy(s, out_hbm.at[pl.ds(0, _CHUNK)], sem_o).wait()

    def compute(bufs):
        x_r, y_r, z_r, p0, p1, p2, _, _ = bufs

        @plsc.parallel_loop(0, _CHUNK, step=_L, unroll=2)
        def _grp(g):
            s = pl.multiple_of(g, _L)

            def coord(v):
                cc = jnp.minimum(jnp.maximum(v * (_GRID - 1.0), 0.0), _CMAX)
                i0 = cc.astype(jnp.int32)
                w = cc - i0.astype(jnp.float32)
                return i0, w

            x0, wx = coord(x_r[pl.ds(s, _L)])
            y0, wy = coord(y_r[pl.ds(s, _L)])
            z0, wz = coord(z_r[pl.ds(s, _L)])
            i000 = (z0 * _GRID + y0) * _GRID + x0
            idx = [i000 + k if k else i000 for k in _SHIFTS]

            for ref, ob in ((lut0, p0), (lut1, p1), (lut2, p2)):
                c000 = plsc.load_gather(ref, [idx[0]])
                c001 = plsc.load_gather(ref, [idx[1]])
                c010 = plsc.load_gather(ref, [idx[2]])
                c011 = plsc.load_gather(ref, [idx[3]])
                c100 = plsc.load_gather(ref, [idx[4]])
                c101 = plsc.load_gather(ref, [idx[5]])
                c110 = plsc.load_gather(ref, [idx[6]])
                c111 = plsc.load_gather(ref, [idx[7]])
                c00 = c000 + wx * (c001 - c000)
                c01 = c010 + wx * (c011 - c010)
                c10 = c100 + wx * (c101 - c100)
                c11 = c110 + wx * (c111 - c110)
                c0 = c00 + wy * (c01 - c00)
                c1 = c10 + wy * (c11 - c10)
                ob[pl.ds(s, _L)] = c0 + wz * (c1 - c0)

    start_in(set_a, 0)
    for d in (lut0, lut1, lut2):
        pltpu.make_async_copy(lut_hbm.at[pl.ds(0, _NLUT_PAD)], d, sem_lut).wait()

    @pl.loop(0, _NCHUNK, step=2)
    def _pair(t):
        start_in(set_b, t + 1)
        wait_in(set_a)

        @pl.when(t >= 2)
        def _():
            wait_out(set_a)

        compute(set_a)
        start_out(set_a, t)

        @pl.when(t + 2 < _NCHUNK)
        def _():
            start_in(set_a, t + 2)

        wait_in(set_b)

        @pl.when(t >= 2)
        def _():
            wait_out(set_b)

        compute(set_b)
        start_out(set_b, t + 1)

    wait_out(set_a)
    wait_out(set_b)


def kernel(img, lut):
    imgf = img.reshape(_B * _C * _NPIX)
    lutf = lut.reshape(_B, _C, _NLUT)
    lutp = jnp.pad(lutf, ((0, 0), (0, 0), (0, _NLUT_PAD - _NLUT)))
    lutp = lutp.reshape(_B * _C * _NLUT_PAD)
    vm = lambda n: pltpu.VMEM((n,), jnp.float32)
    k = pl.kernel(
        _sc_body,
        out_type=jax.ShapeDtypeStruct((_B * _C * _NPIX,), jnp.float32),
        mesh=plsc.VectorSubcoreMesh(core_axis_name="c", subcore_axis_name="s"),
        scratch_types=[vm(_NLUT_PAD), vm(_NLUT_PAD), vm(_NLUT_PAD)]
                      + [vm(_CHUNK)] * 12
                      + [pltpu.SemaphoreType.DMA] * 5,
        compiler_params=pltpu.CompilerParams(needs_layout_passes=False),
    )
    return k(imgf, lutp).reshape(_B, _C, 512, 512)
